# baseline (speedup 1.0000x reference)
import jax
import jax.numpy as jnp
from jax.experimental import pallas as pl
from jax.experimental.pallas import tpu as pltpu

_NSPLIT = 5


def _scale_body(ei_ref, aw_ref, x_ref, o_ref, w_ref):
    m = ei_ref.shape[0]
    num_l = aw_ref.shape[0]

    @pl.when((pl.program_id(0) == 0) & (pl.program_id(1) == 0))
    def _():
        ei = ei_ref[...]
        onehot = (ei == jax.lax.broadcasted_iota(jnp.int32, (m, num_l), 1))
        w_ref[...] = jax.lax.dot_general(
            onehot.astype(jnp.float32), aw_ref[...],
            (((1,), (0,)), ((), ())),
            preferred_element_type=jnp.float32)

    i = pl.program_id(0)
    o_ref[...] = x_ref[...] * w_ref[pl.ds(i, 1), :][None]


def kernel(node_input, affine_weight, expand_index):
    n, m, c = node_input.shape
    x_t = jnp.transpose(node_input, (1, 0, 2))
    aw = affine_weight.reshape(affine_weight.shape[-2], c)
    ei = expand_index.astype(jnp.int32).reshape(m, 1)

    bn = n // _NSPLIT
    out_t = pl.pallas_call(
        _scale_body,
        grid=(m, _NSPLIT),
        in_specs=[
            pl.BlockSpec((m, 1), lambda i, j: (0, 0)),
            pl.BlockSpec(aw.shape, lambda i, j: (0, 0)),
            pl.BlockSpec((1, bn, c), lambda i, j: (i, j, 0)),
        ],
        out_specs=pl.BlockSpec((1, bn, c), lambda i, j: (i, j, 0)),
        out_shape=jax.ShapeDtypeStruct((m, n, c), jnp.float32),
        scratch_shapes=[pltpu.VMEM((m, c), jnp.float32)],
    )(ei, aw, x_t)
    return jnp.transpose(out_t, (1, 0, 2))


# speedup vs baseline: 1.5553x; 1.5553x over previous
import jax
import jax.numpy as jnp
from jax.experimental import pallas as pl
from jax.experimental.pallas import tpu as pltpu

_NSPLIT = 1


def _scale_body(ei_ref, aw_ref, x_ref, o_ref, w_ref):
    m = ei_ref.shape[0]
    num_l = aw_ref.shape[0]

    @pl.when((pl.program_id(0) == 0) & (pl.program_id(1) == 0))
    def _():
        ei = ei_ref[...]
        onehot = (ei == jax.lax.broadcasted_iota(jnp.int32, (m, num_l), 1))
        w_ref[...] = jax.lax.dot_general(
            onehot.astype(jnp.float32), aw_ref[...],
            (((1,), (0,)), ((), ())),
            preferred_element_type=jnp.float32)

    i = pl.program_id(0)
    o_ref[...] = x_ref[...] * w_ref[pl.ds(i, 1), :][None]


def kernel(node_input, affine_weight, expand_index):
    n, m, c = node_input.shape
    x_t = jnp.transpose(node_input, (1, 0, 2))
    aw = affine_weight.reshape(affine_weight.shape[-2], c)
    ei = expand_index.astype(jnp.int32).reshape(m, 1)

    bn = n // _NSPLIT
    out_t = pl.pallas_call(
        _scale_body,
        grid=(m, _NSPLIT),
        in_specs=[
            pl.BlockSpec((m, 1), lambda i, j: (0, 0)),
            pl.BlockSpec(aw.shape, lambda i, j: (0, 0)),
            pl.BlockSpec((1, bn, c), lambda i, j: (i, j, 0)),
        ],
        out_specs=pl.BlockSpec((1, bn, c), lambda i, j: (i, j, 0)),
        out_shape=jax.ShapeDtypeStruct((m, n, c), jnp.float32),
        scratch_shapes=[pltpu.VMEM((m, c), jnp.float32)],
    )(ei, aw, x_t)
    return jnp.transpose(out_t, (1, 0, 2))


# speedup vs baseline: 1.5762x; 1.0135x over previous
import jax
import jax.numpy as jnp
from jax.experimental import pallas as pl
from jax.experimental.pallas import tpu as pltpu

_MBLK = 2


def _scale_body(ei_ref, aw_ref, x_ref, o_ref, w_ref):
    m = ei_ref.shape[0]
    num_l = aw_ref.shape[0]

    @pl.when(pl.program_id(0) == 0)
    def _():
        ei = ei_ref[...]
        onehot = (ei == jax.lax.broadcasted_iota(jnp.int32, (m, num_l), 1))
        w_ref[pl.ds(0, m), :] = jax.lax.dot_general(
            onehot.astype(jnp.float32), aw_ref[...],
            (((1,), (0,)), ((), ())),
            preferred_element_type=jnp.float32)

    i = pl.program_id(0)
    o_ref[...] = x_ref[...] * w_ref[pl.ds(i * _MBLK, _MBLK), :][:, None, :]


def kernel(node_input, affine_weight, expand_index):
    n, m, c = node_input.shape
    x_t = jnp.transpose(node_input, (1, 0, 2))
    aw = affine_weight.reshape(affine_weight.shape[-2], c)
    ei = expand_index.astype(jnp.int32).reshape(m, 1)

    mb = _MBLK
    out_t = pl.pallas_call(
        _scale_body,
        grid=((m + mb - 1) // mb,),
        in_specs=[
            pl.BlockSpec((m, 1), lambda i: (0, 0)),
            pl.BlockSpec(aw.shape, lambda i: (0, 0)),
            pl.BlockSpec((mb, n, c), lambda i: (i, 0, 0)),
        ],
        out_specs=pl.BlockSpec((mb, n, c), lambda i: (i, 0, 0)),
        out_shape=jax.ShapeDtypeStruct((m, n, c), jnp.float32),
        scratch_shapes=[pltpu.VMEM((m + m % mb, c), jnp.float32)],
    )(ei, aw, x_t)
    return jnp.transpose(out_t, (1, 0, 2))
